# Initial kernel scaffold; baseline (speedup 1.0000x reference)
import jax
import jax.numpy as jnp
from jax.experimental import pallas as pl


def kernel(scores):
    raise NotImplementedError("write your pallas kernel here")


# speedup vs baseline: 27.7671x; 27.7671x over previous
import jax
import jax.numpy as jnp
from jax.experimental import pallas as pl

_K = 256
_ROWS_PER_BLOCK = 16


def _topk_mask_kernel(x_ref, o_ref):
    x = x_ref[...]
    bits = jax.lax.bitcast_convert_type(x, jnp.uint32)
    ukey = jnp.where(
        bits >= jnp.uint32(0x80000000), ~bits, bits | jnp.uint32(0x80000000)
    )
    rows = x.shape[0]
    p = jnp.zeros((rows, 1), jnp.uint32)
    for b in range(31, -1, -1):
        t = p | jnp.uint32(1 << b)
        cnt = jnp.sum((ukey >= t).astype(jnp.int32), axis=1, keepdims=True)
        p = jnp.where(cnt >= _K, t, p)
    o_ref[...] = (ukey >= p).astype(jnp.float32)


def kernel(scores):
    b, n = scores.shape
    return pl.pallas_call(
        _topk_mask_kernel,
        grid=(b // _ROWS_PER_BLOCK,),
        in_specs=[pl.BlockSpec((_ROWS_PER_BLOCK, n), lambda i: (i, 0))],
        out_specs=pl.BlockSpec((_ROWS_PER_BLOCK, n), lambda i: (i, 0)),
        out_shape=jax.ShapeDtypeStruct((b, n), jnp.float32),
    )(scores)
